# baseline (speedup 1.0000x reference)
import functools
import math

import jax
import jax.numpy as jnp
from jax import lax
from jax.experimental import pallas as pl
from jax.experimental.pallas import tpu as pltpu
from jax.experimental.pallas import tpu_sc as plsc


def _pos_encoding(max_len, embed_dim):
    idx = jnp.arange(0, embed_dim, 2, dtype=jnp.float32)
    pos = jnp.arange(0, max_len, dtype=jnp.float32)[:, None]
    div_term = jnp.exp(-idx / embed_dim * math.log(10000.0))
    ang = pos * div_term
    pe = jnp.zeros((max_len, embed_dim), dtype=jnp.float32)
    pe = pe.at[:, 0::2].set(jnp.sin(ang))
    pe = pe.at[:, 1::2].set(jnp.cos(ang))
    return pe


@functools.lru_cache(maxsize=None)
def _build_sc_kernel(B, L, V, D):
    info = plsc.get_sparse_core_info()
    NC, NS = info.num_cores, info.num_subcores
    NW = NC * NS
    assert B % (NW * 128) == 0 and D % 8 == 0
    SPW = B // NW
    assert SPW == 128
    LC = 2
    assert L % LC == 0
    G = L // LC
    assert G % 2 == 0 and G >= 4
    N = LC * 128
    scale = math.sqrt(D)
    pieces = [(o, min(128, N - o)) for o in range(0, N, 128)]

    mesh = plsc.VectorSubcoreMesh(core_axis_name="c", subcore_axis_name="s")

    @functools.partial(
        pl.kernel,
        out_type=jax.ShapeDtypeStruct((L, D // 8, NW, 8, 128), jnp.float32),
        mesh=mesh,
        scratch_types=[
            pltpu.VMEM((SPW * L,), jnp.int32),
            pltpu.VMEM((N,), jnp.int32),
            pltpu.VMEM((N,), jnp.int32),
            pltpu.VMEM((L * D,), jnp.float32),
            pltpu.VMEM((N, D), jnp.float32),
            pltpu.VMEM((N, D), jnp.float32),
            pltpu.VMEM((LC, D // 8, 8, 128), jnp.float32),
            pltpu.VMEM((LC, D // 8, 8, 128), jnp.float32),
            pltpu.SemaphoreType.DMA,
            pltpu.SemaphoreType.DMA,
            pltpu.SemaphoreType.DMA,
            pltpu.SemaphoreType.DMA,
        ],
        compiler_params=pltpu.CompilerParams(
            use_tc_tiling_on_sc=False,
            needs_layout_passes=False,
            skip_device_barrier=True,
        ),
    )
    def sc_embed(x_hbm, pe_hbm, w_hbm, out_hbm,
                 xb, ib0, ib1, pe_v, r0, r1, t0, t1, sg0, sg1, so0, so1):
        wid = lax.axis_index("s") * NC + lax.axis_index("c")
        ib, rows, tb = [ib0, ib1], [r0, r1], [t0, t1]
        sg, so = [sg0, sg1], [so0, so1]

        pltpu.sync_copy(pe_hbm, pe_v)
        pltpu.sync_copy(x_hbm.at[pl.ds(wid * SPW * L, SPW * L)], xb)

        iota16 = lax.iota(jnp.int32, 16)
        iota_l = iota16 * L

        def build_idx(gg, p):
            for l in range(LC):
                for bg in range(8):
                    iv = iota_l + (bg * 16 * L + gg * LC + l)
                    ib[p][pl.ds(l * 128 + bg * 16, 16)] = plsc.load_gather(
                        xb, [iv]
                    )

        def gather_descs(p):
            return [
                pltpu.make_async_copy(
                    w_hbm.at[ib[p].at[pl.ds(o, n)]],
                    rows[p].at[pl.ds(o, n)],
                    sg[p],
                )
                for o, n in pieces
            ]

        def out_desc(gg, p):
            return pltpu.make_async_copy(
                tb[p], out_hbm.at[pl.ds(gg * LC, LC), :, wid], so[p]
            )

        def compute(gg, p):
            rp, tp = rows[p], tb[p]
            for l in range(LC):
                rvecs = [iota16 + (l * 128 + bg * 16) for bg in range(8)]
                bvecs = [iota16 + bg * 16 for bg in range(8)]
                lsplat = jnp.full((16,), l, jnp.int32)

                @pl.loop(0, D, unroll=2)
                def _j(j):
                    cols = (iota16 + j) & (D - 1)
                    jt_v = lax.shift_right_logical(cols, 3)
                    js_v = cols & 7
                    pe_b = plsc.load_gather(pe_v, [cols + (gg * LC + l) * D])
                    vals = [
                        plsc.load_gather(rp, [rvecs[bg], cols])
                        for bg in range(8)
                    ]
                    for bg in range(8):
                        plsc.store_scatter(
                            tp,
                            [lsplat, jt_v, js_v, bvecs[bg]],
                            vals[bg] * scale + pe_b,
                        )

        build_idx(0, 0)
        for d in gather_descs(0):
            d.start()

        @pl.loop(0, G, step=2)
        def _it(g):
            for p in (0, 1):
                gg = g + p
                for d in gather_descs(p):
                    d.wait()

                @pl.when(gg < G - 1)
                def _():
                    build_idx(gg + 1, 1 - p)
                    for d in gather_descs(1 - p):
                        d.start()

                @pl.when(gg >= 2)
                def _():
                    out_desc(gg - 2, p).wait()

                compute(gg, p)
                out_desc(gg, p).start()

        out_desc(G - 2, 0).wait()
        out_desc(G - 1, 1).wait()

    return sc_embed


def kernel(x, W):
    B, L = x.shape
    V, D = W.shape
    pe = _pos_encoding(L, D)
    sc_embed = _build_sc_kernel(B, L, V, D)
    out5 = sc_embed(x.reshape(B * L), pe.reshape(L * D), W)
    return out5.transpose(2, 4, 0, 1, 3).reshape(B, L, D)


# speedup vs baseline: 1.0028x; 1.0028x over previous
import functools
import math

import jax
import jax.numpy as jnp
from jax import lax
from jax.experimental import pallas as pl
from jax.experimental.pallas import tpu as pltpu
from jax.experimental.pallas import tpu_sc as plsc


def _pos_encoding(max_len, embed_dim):
    idx = jnp.arange(0, embed_dim, 2, dtype=jnp.float32)
    pos = jnp.arange(0, max_len, dtype=jnp.float32)[:, None]
    div_term = jnp.exp(-idx / embed_dim * math.log(10000.0))
    ang = pos * div_term
    pe = jnp.zeros((max_len, embed_dim), dtype=jnp.float32)
    pe = pe.at[:, 0::2].set(jnp.sin(ang))
    pe = pe.at[:, 1::2].set(jnp.cos(ang))
    return pe


@functools.lru_cache(maxsize=None)
def _build_sc_kernel(B, L, V, D):
    info = plsc.get_sparse_core_info()
    NC, NS = info.num_cores, info.num_subcores
    NW = NC * NS
    assert B % (NW * 128) == 0 and D % 8 == 0
    SPW = B // NW
    assert SPW == 128
    LC = 2
    assert L % LC == 0
    G = L // LC
    assert G % 2 == 0 and G >= 4
    N = LC * 128
    scale = math.sqrt(D)
    pieces = [(o, min(128, N - o)) for o in range(0, N, 128)]

    mesh = plsc.VectorSubcoreMesh(core_axis_name="c", subcore_axis_name="s")

    @functools.partial(
        pl.kernel,
        out_type=jax.ShapeDtypeStruct((L, D // 8, NW, 8, 128), jnp.float32),
        mesh=mesh,
        scratch_types=[
            pltpu.VMEM((SPW * L,), jnp.int32),
            pltpu.VMEM((N,), jnp.int32),
            pltpu.VMEM((N,), jnp.int32),
            pltpu.VMEM((L * D,), jnp.float32),
            pltpu.VMEM((N, D), jnp.float32),
            pltpu.VMEM((N, D), jnp.float32),
            pltpu.VMEM((LC, D // 8, 8, 128), jnp.float32),
            pltpu.VMEM((LC, D // 8, 8, 128), jnp.float32),
            pltpu.SemaphoreType.DMA,
            pltpu.SemaphoreType.DMA,
            pltpu.SemaphoreType.DMA,
            pltpu.SemaphoreType.DMA,
        ],
        compiler_params=pltpu.CompilerParams(
            use_tc_tiling_on_sc=False,
            needs_layout_passes=False,
            skip_device_barrier=True,
            has_side_effects=False,
        ),
    )
    def sc_embed(x_hbm, pe_hbm, w_hbm, out_hbm,
                 xb, ib0, ib1, pe_v, r0, r1, t0, t1, sg0, sg1, so0, so1):
        wid = lax.axis_index("s") * NC + lax.axis_index("c")
        ib, rows, tb = [ib0, ib1], [r0, r1], [t0, t1]
        sg, so = [sg0, sg1], [so0, so1]

        pltpu.sync_copy(pe_hbm, pe_v)
        pltpu.sync_copy(x_hbm.at[pl.ds(wid * SPW * L, SPW * L)], xb)

        iota16 = lax.iota(jnp.int32, 16)
        iota_l = iota16 * L

        def build_idx(gg, p):
            for l in range(LC):
                for bg in range(8):
                    iv = iota_l + (bg * 16 * L + gg * LC + l)
                    ib[p][pl.ds(l * 128 + bg * 16, 16)] = plsc.load_gather(
                        xb, [iv]
                    )

        def gather_descs(p):
            return [
                pltpu.make_async_copy(
                    w_hbm.at[ib[p].at[pl.ds(o, n)]],
                    rows[p].at[pl.ds(o, n)],
                    sg[p],
                )
                for o, n in pieces
            ]

        def out_desc(gg, p):
            return pltpu.make_async_copy(
                tb[p], out_hbm.at[pl.ds(gg * LC, LC), :, wid], so[p]
            )

        def compute(gg, p):
            rp, tp = rows[p], tb[p]
            for l in range(LC):
                rvecs = [iota16 + (l * 128 + bg * 16) for bg in range(8)]
                bvecs = [iota16 + bg * 16 for bg in range(8)]
                lsplat = jnp.full((16,), l, jnp.int32)

                @pl.loop(0, D, unroll=2)
                def _j(j):
                    cols = (iota16 + j) & (D - 1)
                    jt_v = lax.shift_right_logical(cols, 3)
                    js_v = cols & 7
                    pe_b = plsc.load_gather(pe_v, [cols + (gg * LC + l) * D])
                    vals = [
                        plsc.load_gather(rp, [rvecs[bg], cols])
                        for bg in range(8)
                    ]
                    for bg in range(8):
                        plsc.store_scatter(
                            tp,
                            [lsplat, jt_v, js_v, bvecs[bg]],
                            vals[bg] * scale + pe_b,
                        )

        build_idx(0, 0)
        for d in gather_descs(0):
            d.start()

        @pl.loop(0, G, step=2)
        def _it(g):
            for p in (0, 1):
                gg = g + p
                for d in gather_descs(p):
                    d.wait()

                @pl.when(gg < G - 1)
                def _():
                    build_idx(gg + 1, 1 - p)
                    for d in gather_descs(1 - p):
                        d.start()

                @pl.when(gg >= 2)
                def _():
                    out_desc(gg - 2, p).wait()

                compute(gg, p)
                out_desc(gg, p).start()

        out_desc(G - 2, 0).wait()
        out_desc(G - 1, 1).wait()

    return sc_embed


def kernel(x, W):
    B, L = x.shape
    V, D = W.shape
    pe = _pos_encoding(L, D)
    sc_embed = _build_sc_kernel(B, L, V, D)
    out5 = sc_embed(x.reshape(B * L), pe.reshape(L * D), W)
    return out5.transpose(2, 4, 0, 1, 3).reshape(B, L, D)
